# baseline (speedup 1.0000x reference)
import jax
import jax.numpy as jnp
from jax import lax
from jax.experimental import pallas as pl
from jax.experimental.pallas import tpu as pltpu

_SIZE = 65536
_VALID = 40001
_TILE = 1024
_NTILE = _SIZE // _TILE
_NFULL = _VALID // _TILE
_TAILN = _VALID - _NFULL * _TILE


def _body(ptr_ref, nv_ref, hn_ref, hist_ref, out_ref, thr_ref, buf, sem):
    idx = ptr_ref[0, 0] % _SIZE
    nv = nv_ref[0, 0]
    halfnoise = hn_ref[0, 0]

    s_v = jnp.zeros((_TILE,), jnp.float32)
    q_v = jnp.zeros((_TILE,), jnp.float32)
    tmask = lax.broadcasted_iota(jnp.int32, (_TILE,), 0) < _TAILN
    for t in range(_NTILE):
        v = hist_ref[pl.ds(t * _TILE, _TILE)]
        buf[pl.ds(t * _TILE, _TILE)] = v
        if t < _NFULL:
            s_v = s_v + v
            q_v = q_v + v * v
        elif t == _NFULL:
            vm = jnp.where(tmask, v, 0.0)
            s_v = s_v + vm
            q_v = q_v + vm * vm

    base = pl.multiple_of((idx // 128) * 128, 128)
    off = idx % 128
    blk = buf[pl.ds(base, 128)]
    sel = lax.broadcasted_iota(jnp.int32, (128,), 0) == off
    buf[pl.ds(base, 128)] = jnp.where(sel, nv, blk)

    cp = pltpu.async_copy(buf, out_ref, sem)

    s = jnp.sum(s_v)
    q = jnp.sum(q_v)
    old = jnp.sum(jnp.where(sel, blk, 0.0))
    inb = (idx < _VALID).astype(jnp.float32)
    s = s + inb * (nv - old)
    q = q + inb * (nv * nv - old * old)

    inv_n = jnp.float32(1.0 / _VALID)
    mean = s * inv_n
    var = jnp.maximum(q * inv_n - mean * mean, 0.0)
    std = jnp.sqrt(var)
    thr_ref[0, 0] = mean + halfnoise * std

    cp.wait()


_call = pl.pallas_call(
    _body,
    out_shape=(
        jax.ShapeDtypeStruct((_SIZE,), jnp.float32),
        jax.ShapeDtypeStruct((1, 1), jnp.float32),
    ),
    in_specs=[
        pl.BlockSpec(memory_space=pltpu.SMEM),
        pl.BlockSpec(memory_space=pltpu.SMEM),
        pl.BlockSpec(memory_space=pltpu.SMEM),
        pl.BlockSpec(memory_space=pltpu.VMEM),
    ],
    out_specs=(
        pl.BlockSpec(memory_space=pl.ANY),
        pl.BlockSpec(memory_space=pltpu.SMEM),
    ),
    scratch_shapes=[
        pltpu.VMEM((_SIZE,), jnp.float32),
        pltpu.SemaphoreType.DMA,
    ],
)


@jax.jit
def kernel(history, new_value, pointer):
    ptr = jnp.asarray(pointer, jnp.int32).reshape(1, 1)
    nv = jnp.asarray(new_value, jnp.float32).reshape(1, 1)
    noise = jax.random.normal(jax.random.key(42), (), dtype=jnp.float32)
    hn = (noise * jnp.float32(0.5)).reshape(1, 1)
    upd, thr = _call(ptr, nv, hn, history)
    return upd, thr[0, 0]
